# Initial kernel scaffold; baseline (speedup 1.0000x reference)
import jax
import jax.numpy as jnp
from jax.experimental import pallas as pl


def kernel(indices, table):
    raise NotImplementedError("write your pallas kernel here")


# speedup vs baseline: 1.3071x; 1.3071x over previous
import functools

import jax
import jax.numpy as jnp
from jax import lax
from jax.experimental import pallas as pl
from jax.experimental.pallas import tpu as pltpu
from jax.experimental.pallas import tpu_sc as plsc

VOCAB = 1000000
EMBED_DIM = 32
BATCH = 4096
HIST = 200

NC = 2
NS = 16
NW = NC * NS

TOTAL = BATCH * HIST
CHUNK = 128
PER_W = TOTAL // NW
NCHUNK = PER_W // CHUNK


def _make_kernel():
  mesh = plsc.VectorSubcoreMesh(
      core_axis_name="c", subcore_axis_name="s", num_cores=NC, num_subcores=NS
  )

  @functools.partial(
      pl.kernel,
      out_type=jax.ShapeDtypeStruct((NW, NCHUNK, CHUNK, EMBED_DIM), jnp.float32),
      mesh=mesh,
      scratch_types=[
          pltpu.VMEM((NCHUNK, CHUNK), jnp.int32),
          pltpu.VMEM((CHUNK, EMBED_DIM), jnp.float32),
          pltpu.SemaphoreType.DMA,
      ],
      compiler_params=pltpu.CompilerParams(use_tc_tiling_on_sc=False),
  )
  def gather_kernel(idx_hbm, table_hbm, out_hbm, idx_v, rows_v, sem):
    wid = lax.axis_index("s") * NC + lax.axis_index("c")
    pltpu.sync_copy(idx_hbm.at[wid], idx_v)

    def step(j, carry):
      pltpu.async_copy(table_hbm.at[idx_v.at[j]], rows_v, sem).wait()
      pltpu.sync_copy(rows_v, out_hbm.at[wid].at[j])
      return carry

    lax.fori_loop(0, NCHUNK, step, 0)

  return gather_kernel


_gather = _make_kernel()


@jax.jit
def kernel(indices, table):
  idx = indices.astype(jnp.int32).reshape(NW, NCHUNK, CHUNK)
  out = _gather(idx, table)
  return out.reshape(BATCH, HIST, EMBED_DIM)


# speedup vs baseline: 1.5038x; 1.1505x over previous
import functools

import jax
import jax.numpy as jnp
from jax import lax
from jax.experimental import pallas as pl
from jax.experimental.pallas import tpu as pltpu
from jax.experimental.pallas import tpu_sc as plsc

VOCAB = 1000000
EMBED_DIM = 32
BATCH = 4096
HIST = 200

NC = 2
NS = 16
NW = NC * NS

TOTAL = BATCH * HIST
CHUNK = 128
PER_W = TOTAL // NW
NCHUNK = PER_W // CHUNK


K = 8
NGROUP = NCHUNK // K


def _make_kernel():
  mesh = plsc.VectorSubcoreMesh(
      core_axis_name="c", subcore_axis_name="s", num_cores=NC, num_subcores=NS
  )

  @functools.partial(
      pl.kernel,
      out_type=jax.ShapeDtypeStruct((NW, NCHUNK, CHUNK, EMBED_DIM), jnp.float32),
      mesh=mesh,
      scratch_types=[
          pltpu.VMEM((NCHUNK, CHUNK), jnp.int32),
          pltpu.VMEM((2, K, CHUNK, EMBED_DIM), jnp.float32),
          pltpu.SemaphoreType.DMA((2,)),
          pltpu.SemaphoreType.DMA((2,)),
      ],
      compiler_params=pltpu.CompilerParams(use_tc_tiling_on_sc=False),
  )
  def gather_kernel(idx_hbm, table_hbm, out_hbm, idx_v, rows_v, sem_g, sem_o):
    wid = lax.axis_index("s") * NC + lax.axis_index("c")
    pltpu.sync_copy(idx_hbm.at[wid], idx_v)
    my_out = out_hbm.at[wid]

    def fire_gathers(g, slot):
      base = g * K
      for b in range(K):
        pltpu.async_copy(
            table_hbm.at[idx_v.at[base + b]], rows_v.at[slot, b], sem_g.at[slot]
        )

    def wait_gathers(slot):
      for b in range(K):
        pltpu.make_async_copy(
            table_hbm.at[idx_v.at[b]], rows_v.at[slot, b], sem_g.at[slot]
        ).wait()

    def fire_wb(g, slot):
      pltpu.async_copy(
          rows_v.at[slot], my_out.at[pl.ds(g * K, K)], sem_o.at[slot]
      )

    def wait_wb(g, slot):
      pltpu.make_async_copy(
          rows_v.at[slot], my_out.at[pl.ds(g * K, K)], sem_o.at[slot]
      ).wait()

    fire_gathers(0, 0)

    def body(g, carry):
      slot = lax.rem(g, 2)
      pl.when(g >= 2)(lambda: wait_wb(g - 2, slot))
      fire_gathers(g, slot)
      wait_gathers(1 - slot)
      fire_wb(g - 1, 1 - slot)
      return carry

    lax.fori_loop(1, NGROUP, body, 0)

    last = NGROUP - 1
    last_slot = last % 2
    wait_gathers(last_slot)
    fire_wb(last, last_slot)
    wait_wb(last - 1, 1 - last_slot)
    wait_wb(last, last_slot)

  return gather_kernel


_gather = _make_kernel()


@jax.jit
def kernel(indices, table):
  idx = indices.astype(jnp.int32).reshape(NW, NCHUNK, CHUNK)
  out = _gather(idx, table)
  return out.reshape(BATCH, HIST, EMBED_DIM)
